# baseline (speedup 1.0000x reference)
import jax
import jax.numpy as jnp
from jax.experimental import pallas as pl

N0 = 100000
N1 = 160000
N2 = 20000
NG = 2048
L = 3
H = 64
OUT = 10


def _bn(x, g, b):
    return x * g + b


def _mlp2(x, W, b, g, bt):
    for i in range(2):
        x = jax.nn.relu(_bn(x @ W[i] + b[i], g[i], bt[i]))
    return x


def _final_kernel(x_ref, w_ref, b_ref, o_ref):
    o_ref[...] = x_ref[...] @ w_ref[...] + b_ref[...]


def kernel(x0, up_index_0, up_index_1, boundary_index_1, boundary_index_2,
           batch0, batch1, batch2, atom_emb, W_up, b_up, g_up, bt_up,
           W_bd, b_bd, g_bd, bt_bd, W_cmb, b_cmb, g_cmb, bt_cmb,
           W1, b1, g1, bt1, W2, b2):
    x0f = jnp.zeros((N0, H), jnp.float32)
    for fidx in range(9):
        x0f = x0f + atom_emb[fidx][x0[:, fidx]]
    x1f = jax.ops.segment_sum(x0f[boundary_index_1[0]], boundary_index_1[1], num_segments=N1)
    x2f = jax.ops.segment_sum(x1f[boundary_index_2[0]], boundary_index_2[1], num_segments=N2)
    xs = [x0f, x1f, x2f]
    sizes = [N0, N1, N2]
    ups = [up_index_0, up_index_1, None]
    bds = [None, boundary_index_1, boundary_index_2]
    for l in range(L):
        new_xs = []
        for d in range(3):
            x = xs[d]
            if ups[d] is not None:
                up_msg = jax.ops.segment_sum(x[ups[d][0]], ups[d][1], num_segments=sizes[d])
            else:
                up_msg = jnp.zeros_like(x)
            if bds[d] is not None:
                bd_msg = jax.ops.segment_sum(xs[d - 1][bds[d][0]], bds[d][1], num_segments=sizes[d])
            else:
                bd_msg = jnp.zeros_like(x)
            h_up = _mlp2(x + up_msg, W_up[l, d], b_up[l, d], g_up[l, d], bt_up[l, d])
            h_bd = _mlp2(x + bd_msg, W_bd[l, d], b_bd[l, d], g_bd[l, d], bt_bd[l, d])
            h = jnp.concatenate([h_up, h_bd], axis=-1)
            h = jax.nn.relu(_bn(h @ W_cmb[l, d] + b_cmb[l, d], g_cmb[l, d], bt_cmb[l, d]))
            new_xs.append(h)
        xs = new_xs
    batches = [batch0, batch1, batch2]
    pooled = [jax.ops.segment_sum(xs[d], batches[d], num_segments=NG) for d in range(3)]
    outs = [jax.nn.relu(_bn(pooled[d] @ W1[d] + b1[d], g1[d], bt1[d])) for d in range(3)]
    x = outs[0] + outs[1] + outs[2]
    return pl.pallas_call(
        _final_kernel,
        out_shape=jax.ShapeDtypeStruct((NG, OUT), jnp.float32),
    )(x, W2, b2)


# speedup vs baseline: 3.3249x; 3.3249x over previous
import functools

import jax
import jax.numpy as jnp
from jax import lax
from jax.experimental import pallas as pl
from jax.experimental.pallas import tpu as pltpu
from jax.experimental.pallas import tpu_sc as plsc

N0 = 100000
N1 = 160000
N2 = 20000
NG = 2048
L = 3
H = 64
OUT = 10

NB = 128
BIG = 1 << 29

CFG0 = (6, 16896)
CFG1 = (8, 20480)
CFG2 = (2, 10240)
CFGG = (2, 1024)
CMAX = 20480

NP0 = CFG0[0] * CFG0[1]
NP1 = CFG1[0] * CFG1[1]
NP2 = CFG2[0] * CFG2[1]


@functools.lru_cache(maxsize=None)
def _make_spmm(e_pad, nc, C, n_src):
    out_rows = nc * C
    mesh = plsc.VectorSubcoreMesh(core_axis_name="c", subcore_axis_name="s",
                                  num_cores=2, num_subcores=16)
    wr = C // 16

    def body(tab, srci, dsti, starts, zeros, out,
             sidx, didx, lidx, rows, starts_v, acc, sem, sem2):
        c = lax.axis_index("c")
        t = lax.axis_index("s")
        pltpu.sync_copy(starts, starts_v)
        sv = starts_v[...]
        for k in range(nc):
            @pl.when(c == (k % 2))
            def _chunk(k=k):
                base = k * C
                pltpu.sync_copy(zeros.at[pl.ds(t * wr, wr)],
                                acc.at[pl.ds(t * wr, wr)])
                plsc.subcore_barrier()
                s = sv[k]
                e = sv[k + 1]
                s8 = (s // 8) * 8

                @pl.loop((s8 + t * NB).astype(jnp.int32), e, step=16 * NB)
                def blk(off):
                    off = pl.multiple_of(off, 8)
                    cp1 = pltpu.async_copy(srci.at[pl.ds(off, NB)], sidx, sem)
                    cp2 = pltpu.async_copy(dsti.at[pl.ds(off, NB)], didx, sem2)
                    cp1.wait()
                    cp2.wait()
                    for j in range(NB // 16):
                        dv = didx[pl.ds(j * 16, 16)]
                        lv = dv - base
                        ok = (lv >= 0) & (lv < C)
                        garb = C + lax.iota(jnp.int32, 16)
                        lidx[pl.ds(j * 16, 16)] = jnp.where(ok, lv, garb)
                    pltpu.async_copy(tab.at[sidx], rows, sem).wait()
                    pltpu.async_copy(rows, acc.at[lidx], sem2, add=True).wait()

                plsc.subcore_barrier()
                pltpu.sync_copy(acc.at[pl.ds(t * wr, wr)],
                                out.at[pl.ds(base + t * wr, wr)])

    return pl.kernel(
        body,
        out_type=jax.ShapeDtypeStruct((out_rows, H), jnp.float32),
        mesh=mesh,
        scratch_types=[
            pltpu.VMEM((NB,), jnp.int32),
            pltpu.VMEM((NB,), jnp.int32),
            pltpu.VMEM((NB,), jnp.int32),
            pltpu.VMEM((NB, H), jnp.float32),
            pltpu.VMEM((16,), jnp.int32),
            pltpu.VMEM_SHARED((C + 16, H), jnp.float32),
            pltpu.SemaphoreType.DMA,
            pltpu.SemaphoreType.DMA,
        ],
        compiler_params=pltpu.CompilerParams(use_tc_tiling_on_sc=False),
    )


def _spmm(tab, srci, dsti, starts16, zeros, cfg):
    nc, C = cfg
    k = _make_spmm(srci.shape[0], nc, C, tab.shape[0])
    return k(tab, srci, dsti, starts16, zeros)


def _pad_edges(src, dst, n_src):
    e = src.shape[0]
    e_pad = ((e + NB - 1) // NB + 1) * NB
    npad = e_pad - e
    pad_src = (jnp.arange(npad, dtype=jnp.int32) * 67) % n_src
    src = jnp.concatenate([src.astype(jnp.int32), pad_src])
    dst = jnp.concatenate([dst.astype(jnp.int32),
                           jnp.full((npad,), BIG, jnp.int32)])
    return src, dst


def _group_edges(src, dst, n_src, cfg):
    nc, C = cfg
    e = src.shape[0]
    se = max(131072, e)
    key = (dst // C).astype(jnp.int32)
    if se > e:
        key = jnp.concatenate([key, jnp.full((se - e,), nc, jnp.int32)])
    iota = lax.iota(jnp.int32, se)
    sk, perm = lax.sort((key, iota), num_keys=1)
    perm = perm[:e]
    src_g = jnp.take(src.astype(jnp.int32), perm, mode="clip")
    dst_g = jnp.take(dst.astype(jnp.int32), perm, mode="clip")
    starts = jnp.searchsorted(sk, jnp.arange(nc + 1, dtype=jnp.int32)).astype(jnp.int32)
    starts16 = jnp.concatenate([starts, jnp.full((16 - nc - 1,), e, jnp.int32)])
    src_g, dst_g = _pad_edges(src_g, dst_g, n_src)
    return src_g, dst_g, starts16


def _sorted_starts(dst_sorted, cfg, e):
    nc, C = cfg
    bounds = jnp.arange(nc + 1, dtype=jnp.int32) * C
    starts = jnp.searchsorted(dst_sorted, bounds).astype(jnp.int32)
    return jnp.concatenate([starts, jnp.full((16 - nc - 1,), e, jnp.int32)])


BN = 512


def _mlp_pair_body(x_ref, u_ref, v_ref, wu_ref, bu_ref, wb_ref, bb_ref,
                   wc_ref, bc_ref, o_ref):
    x = x_ref[...]
    a = x + u_ref[...]
    b = x + v_ref[...]
    a = jnp.maximum(jnp.dot(a, wu_ref[0], preferred_element_type=jnp.float32) + bu_ref[0], 0.0)
    a = jnp.maximum(jnp.dot(a, wu_ref[1], preferred_element_type=jnp.float32) + bu_ref[1], 0.0)
    b = jnp.maximum(jnp.dot(b, wb_ref[0], preferred_element_type=jnp.float32) + bb_ref[0], 0.0)
    b = jnp.maximum(jnp.dot(b, wb_ref[1], preferred_element_type=jnp.float32) + bb_ref[1], 0.0)
    h = (jnp.dot(a, wc_ref[0], preferred_element_type=jnp.float32)
         + jnp.dot(b, wc_ref[1], preferred_element_type=jnp.float32) + bc_ref[...])
    o_ref[...] = jnp.maximum(h, 0.0)


def _mlp_single_body(msg_first, x_ref, m_ref, wm_ref, bm_ref, wx_ref, bx_ref,
                     wc_ref, bc_ref, o_ref):
    x = x_ref[...]
    a = x + m_ref[...]
    a = jnp.maximum(jnp.dot(a, wm_ref[0], preferred_element_type=jnp.float32) + bm_ref[0], 0.0)
    a = jnp.maximum(jnp.dot(a, wm_ref[1], preferred_element_type=jnp.float32) + bm_ref[1], 0.0)
    b = jnp.maximum(jnp.dot(x, wx_ref[0], preferred_element_type=jnp.float32) + bx_ref[0], 0.0)
    b = jnp.maximum(jnp.dot(b, wx_ref[1], preferred_element_type=jnp.float32) + bx_ref[1], 0.0)
    if msg_first:
        h = (jnp.dot(a, wc_ref[0], preferred_element_type=jnp.float32)
             + jnp.dot(b, wc_ref[1], preferred_element_type=jnp.float32))
    else:
        h = (jnp.dot(b, wc_ref[0], preferred_element_type=jnp.float32)
             + jnp.dot(a, wc_ref[1], preferred_element_type=jnp.float32))
    o_ref[...] = jnp.maximum(h + bc_ref[...], 0.0)


def _w_spec():
    return pl.BlockSpec((2, H, H), lambda i: (0, 0, 0))


def _b_spec():
    return pl.BlockSpec((2, H), lambda i: (0, 0))


def _row_spec():
    return pl.BlockSpec((BN, H), lambda i: (i, 0))


def _mlp_pair(x, u, v, wu, bu, wb, bb, wc, bc):
    n = x.shape[0]
    return pl.pallas_call(
        _mlp_pair_body,
        grid=(n // BN,),
        in_specs=[_row_spec(), _row_spec(), _row_spec(),
                  _w_spec(), _b_spec(), _w_spec(), _b_spec(),
                  _w_spec(), pl.BlockSpec((1, H), lambda i: (0, 0))],
        out_specs=_row_spec(),
        out_shape=jax.ShapeDtypeStruct((n, H), jnp.float32),
    )(x, u, v, wu, bu, wb, bb, wc, bc)


def _mlp_single(x, m, wm, bm, wx, bx, wc, bc, msg_first):
    n = x.shape[0]
    return pl.pallas_call(
        functools.partial(_mlp_single_body, msg_first),
        grid=(n // BN,),
        in_specs=[_row_spec(), _row_spec(),
                  _w_spec(), _b_spec(), _w_spec(), _b_spec(),
                  _w_spec(), pl.BlockSpec((1, H), lambda i: (0, 0))],
        out_specs=_row_spec(),
        out_shape=jax.ShapeDtypeStruct((n, H), jnp.float32),
    )(x, m, wm, bm, wx, bx, wc, bc)


def _readout_body(p0_ref, p1_ref, p2_ref, w1_ref, b1_ref, w2_ref, b2_ref, o_ref):
    t = jnp.maximum(jnp.dot(p0_ref[...], w1_ref[0], preferred_element_type=jnp.float32) + b1_ref[0], 0.0)
    t = t + jnp.maximum(jnp.dot(p1_ref[...], w1_ref[1], preferred_element_type=jnp.float32) + b1_ref[1], 0.0)
    t = t + jnp.maximum(jnp.dot(p2_ref[...], w1_ref[2], preferred_element_type=jnp.float32) + b1_ref[2], 0.0)
    o_ref[...] = jnp.dot(t, w2_ref[...], preferred_element_type=jnp.float32) + b2_ref[...]


def _readout(p0, p1, p2, w1, b1, w2, b2):
    return pl.pallas_call(
        _readout_body,
        grid=(NG // BN,),
        in_specs=[_row_spec(), _row_spec(), _row_spec(),
                  pl.BlockSpec((3, H, 2 * H), lambda i: (0, 0, 0)),
                  pl.BlockSpec((3, 2 * H), lambda i: (0, 0)),
                  pl.BlockSpec((2 * H, OUT), lambda i: (0, 0)),
                  pl.BlockSpec((1, OUT), lambda i: (0, 0))],
        out_specs=pl.BlockSpec((BN, OUT), lambda i: (i, 0)),
        out_shape=jax.ShapeDtypeStruct((NG, OUT), jnp.float32),
    )(p0, p1, p2, w1, b1, w2, b2)


def kernel(x0, up_index_0, up_index_1, boundary_index_1, boundary_index_2,
           batch0, batch1, batch2, atom_emb, W_up, b_up, g_up, bt_up,
           W_bd, b_bd, g_bd, bt_bd, W_cmb, b_cmb, g_cmb, bt_cmb,
           W1, b1, g1, bt1, W2, b2):
    f32 = jnp.float32
    Wup = W_up * g_up[:, :, :, None, :]
    bup = b_up * g_up + bt_up
    Wbd = W_bd * g_bd[:, :, :, None, :]
    bbd = b_bd * g_bd + bt_bd
    Wc = (W_cmb * g_cmb[:, :, None, :]).reshape(L, 3, 2, H, H)
    bc = (b_cmb * g_cmb + bt_cmb)[:, :, None, :]
    W1f = W1 * g1[:, None, :]
    b1f = (b1 * g1 + bt1)
    b1f2 = b1f
    W2f = W2
    b2f = b2[None, :]

    zeros = jnp.zeros((CMAX + 16, H), f32)

    tab_emb = atom_emb.reshape(9 * 100, H).astype(f32)
    codes = (x0.astype(jnp.int32)
             + jnp.arange(9, dtype=jnp.int32)[None, :] * 100).reshape(-1)
    emb_dst = jnp.repeat(jnp.arange(N0, dtype=jnp.int32), 9)
    emb_src, emb_dst = _pad_edges(codes, emb_dst, 900)
    nc0, C0 = CFG0
    est = [min(9 * C0 * k, 9 * N0) for k in range(nc0 + 1)]
    emb_starts = jnp.array(est + [9 * N0] * (16 - nc0 - 1), jnp.int32)
    x0f = _spmm(tab_emb, emb_src, emb_dst, emb_starts, zeros, CFG0)

    u0s, u0d, u0st = _group_edges(up_index_0[0], up_index_0[1], NP0, CFG0)
    u1s, u1d, u1st = _group_edges(up_index_1[0], up_index_1[1], NP1, CFG1)
    b1s, b1d, b1st = _group_edges(boundary_index_1[0], boundary_index_1[1], NP0, CFG1)
    b2s, b2d, b2st = _group_edges(boundary_index_2[0], boundary_index_2[1], NP1, CFG2)

    x1f = _spmm(x0f, b1s, b1d, b1st, zeros, CFG1)
    x2f = _spmm(x1f, b2s, b2d, b2st, zeros, CFG2)

    xs = [x0f, x1f, x2f]
    for l in range(L):
        up0 = _spmm(xs[0], u0s, u0d, u0st, zeros, CFG0)
        up1 = _spmm(xs[1], u1s, u1d, u1st, zeros, CFG1)
        bm1 = _spmm(xs[0], b1s, b1d, b1st, zeros, CFG1)
        bm2 = _spmm(xs[1], b2s, b2d, b2st, zeros, CFG2)
        x0n = _mlp_single(xs[0], up0, Wup[l, 0], bup[l, 0], Wbd[l, 0], bbd[l, 0],
                          Wc[l, 0], bc[l, 0], msg_first=True)
        x1n = _mlp_pair(xs[1], up1, bm1, Wup[l, 1], bup[l, 1], Wbd[l, 1], bbd[l, 1],
                        Wc[l, 1], bc[l, 1])
        x2n = _mlp_single(xs[2], bm2, Wbd[l, 2], bbd[l, 2], Wup[l, 2], bup[l, 2],
                          Wc[l, 2], bc[l, 2], msg_first=False)
        xs = [x0n, x1n, x2n]

    pools = []
    for d, (bat, n_d, npd) in enumerate(
            [(batch0, N0, NP0), (batch1, N1, NP1), (batch2, N2, NP2)]):
        psrc = jnp.arange(n_d, dtype=jnp.int32)
        pst = _sorted_starts(bat.astype(jnp.int32), CFGG, n_d)
        ps, pd_ = _pad_edges(psrc, bat.astype(jnp.int32), npd)
        pools.append(_spmm(xs[d], ps, pd_, pst, zeros, CFGG))

    return _readout(pools[0], pools[1], pools[2], W1f, b1f2, W2f, b2f)


# speedup vs baseline: 3.4995x; 1.0525x over previous
import functools

import jax
import jax.numpy as jnp
from jax import lax
from jax.experimental import pallas as pl
from jax.experimental.pallas import tpu as pltpu
from jax.experimental.pallas import tpu_sc as plsc

N0 = 100000
N1 = 160000
N2 = 20000
NG = 2048
L = 3
H = 64
OUT = 10

NB = 128
G = 4
RUN = G * NB
BIG = 1 << 29

CFG0 = (6, 16896)
CFG1 = (8, 20480)
CFG2 = (2, 10240)
CFGG = (2, 1024)
CMAX = 20480

NP0 = CFG0[0] * CFG0[1]
NP1 = CFG1[0] * CFG1[1]
NP2 = CFG2[0] * CFG2[1]


@functools.lru_cache(maxsize=None)
def _make_spmm(e_pad, nc, C, n_src):
    out_rows = nc * C
    mesh = plsc.VectorSubcoreMesh(core_axis_name="c", subcore_axis_name="s",
                                  num_cores=2, num_subcores=16)
    wr = C // 16

    def body(tab, srci, dsti, starts, zeros, out,
             sidx, didx, l0, l1, l2, l3, r0, r1, r2, r3,
             starts_v, acc, sem_si, sem_di, g0, g1, g2, g3, sem_sc):
        lidx = [l0, l1, l2, l3]
        rows = [r0, r1, r2, r3]
        gsem = [g0, g1, g2, g3]
        c = lax.axis_index("c")
        t = lax.axis_index("s")
        pltpu.sync_copy(starts, starts_v)
        sv = starts_v[...]
        for k in range(nc):
            @pl.when(c == (k % 2))
            def _chunk(k=k):
                base = k * C
                pltpu.sync_copy(zeros.at[pl.ds(t * wr, wr)],
                                acc.at[pl.ds(t * wr, wr)])
                plsc.subcore_barrier()
                s = sv[k]
                e = sv[k + 1]
                s8 = (s // 8) * 8

                @pl.loop((s8 + t * RUN).astype(jnp.int32), e, step=16 * RUN)
                def blk(off):
                    off = pl.multiple_of(off, 8)
                    cp1 = pltpu.async_copy(srci.at[pl.ds(off, RUN)], sidx, sem_si)
                    cp2 = pltpu.async_copy(dsti.at[pl.ds(off, RUN)], didx, sem_di)
                    cp1.wait()
                    cp2.wait()
                    for g in range(G):
                        for j in range(NB // 16):
                            dv = didx[pl.ds(g * NB + j * 16, 16)]
                            lv = dv - base
                            ok = (lv >= 0) & (lv < C)
                            garb = C + lax.iota(jnp.int32, 16)
                            lidx[g][pl.ds(j * 16, 16)] = jnp.where(ok, lv, garb)
                    cps = [pltpu.async_copy(tab.at[sidx.at[pl.ds(g * NB, NB)]],
                                            rows[g], gsem[g]) for g in range(G)]
                    scs = []
                    for g in range(G):
                        cps[g].wait()
                        scs.append(pltpu.async_copy(rows[g], acc.at[lidx[g]],
                                                    sem_sc, add=True))
                    for sc in scs:
                        sc.wait()

                plsc.subcore_barrier()
                pltpu.sync_copy(acc.at[pl.ds(t * wr, wr)],
                                out.at[pl.ds(base + t * wr, wr)])

    return pl.kernel(
        body,
        out_type=jax.ShapeDtypeStruct((out_rows, H), jnp.float32),
        mesh=mesh,
        scratch_types=(
            [pltpu.VMEM((RUN,), jnp.int32)] * 2
            + [pltpu.VMEM((NB,), jnp.int32)] * G
            + [pltpu.VMEM((NB, H), jnp.float32)] * G
            + [pltpu.VMEM((16,), jnp.int32),
               pltpu.VMEM_SHARED((C + 16, H), jnp.float32)]
            + [pltpu.SemaphoreType.DMA] * (G + 3)
        ),
        compiler_params=pltpu.CompilerParams(use_tc_tiling_on_sc=False),
    )


def _spmm(tab, srci, dsti, starts16, zeros, cfg):
    nc, C = cfg
    k = _make_spmm(srci.shape[0], nc, C, tab.shape[0])
    return k(tab, srci, dsti, starts16, zeros)


def _pad_edges(src, dst, n_src):
    e = src.shape[0]
    e_pad = ((e + RUN - 1) // RUN + 1) * RUN
    npad = e_pad - e
    pad_src = (jnp.arange(npad, dtype=jnp.int32) * 67) % n_src
    src = jnp.concatenate([src.astype(jnp.int32), pad_src])
    dst = jnp.concatenate([dst.astype(jnp.int32),
                           jnp.full((npad,), BIG, jnp.int32)])
    return src, dst


def _group_edges(src, dst, n_src, cfg):
    nc, C = cfg
    e = src.shape[0]
    se = max(131072, e)
    key = (dst // C).astype(jnp.int32)
    if se > e:
        key = jnp.concatenate([key, jnp.full((se - e,), nc, jnp.int32)])
    iota = lax.iota(jnp.int32, se)
    sk, perm = lax.sort((key, iota), num_keys=1)
    perm = perm[:e]
    src_g = jnp.take(src.astype(jnp.int32), perm, mode="clip")
    dst_g = jnp.take(dst.astype(jnp.int32), perm, mode="clip")
    starts = jnp.searchsorted(sk, jnp.arange(nc + 1, dtype=jnp.int32)).astype(jnp.int32)
    starts16 = jnp.concatenate([starts, jnp.full((16 - nc - 1,), e, jnp.int32)])
    src_g, dst_g = _pad_edges(src_g, dst_g, n_src)
    return src_g, dst_g, starts16


def _sorted_starts(dst_sorted, cfg, e):
    nc, C = cfg
    bounds = jnp.arange(nc + 1, dtype=jnp.int32) * C
    starts = jnp.searchsorted(dst_sorted, bounds).astype(jnp.int32)
    return jnp.concatenate([starts, jnp.full((16 - nc - 1,), e, jnp.int32)])


BN = 512


def _mlp_pair_body(x_ref, u_ref, v_ref, wu_ref, bu_ref, wb_ref, bb_ref,
                   wc_ref, bc_ref, o_ref):
    x = x_ref[...]
    a = x + u_ref[...]
    b = x + v_ref[...]
    a = jnp.maximum(jnp.dot(a, wu_ref[0], preferred_element_type=jnp.float32) + bu_ref[0], 0.0)
    a = jnp.maximum(jnp.dot(a, wu_ref[1], preferred_element_type=jnp.float32) + bu_ref[1], 0.0)
    b = jnp.maximum(jnp.dot(b, wb_ref[0], preferred_element_type=jnp.float32) + bb_ref[0], 0.0)
    b = jnp.maximum(jnp.dot(b, wb_ref[1], preferred_element_type=jnp.float32) + bb_ref[1], 0.0)
    h = (jnp.dot(a, wc_ref[0], preferred_element_type=jnp.float32)
         + jnp.dot(b, wc_ref[1], preferred_element_type=jnp.float32) + bc_ref[...])
    o_ref[...] = jnp.maximum(h, 0.0)


def _mlp_single_body(msg_first, x_ref, m_ref, wm_ref, bm_ref, wx_ref, bx_ref,
                     wc_ref, bc_ref, o_ref):
    x = x_ref[...]
    a = x + m_ref[...]
    a = jnp.maximum(jnp.dot(a, wm_ref[0], preferred_element_type=jnp.float32) + bm_ref[0], 0.0)
    a = jnp.maximum(jnp.dot(a, wm_ref[1], preferred_element_type=jnp.float32) + bm_ref[1], 0.0)
    b = jnp.maximum(jnp.dot(x, wx_ref[0], preferred_element_type=jnp.float32) + bx_ref[0], 0.0)
    b = jnp.maximum(jnp.dot(b, wx_ref[1], preferred_element_type=jnp.float32) + bx_ref[1], 0.0)
    if msg_first:
        h = (jnp.dot(a, wc_ref[0], preferred_element_type=jnp.float32)
             + jnp.dot(b, wc_ref[1], preferred_element_type=jnp.float32))
    else:
        h = (jnp.dot(b, wc_ref[0], preferred_element_type=jnp.float32)
             + jnp.dot(a, wc_ref[1], preferred_element_type=jnp.float32))
    o_ref[...] = jnp.maximum(h + bc_ref[...], 0.0)


def _w_spec():
    return pl.BlockSpec((2, H, H), lambda i: (0, 0, 0))


def _b_spec():
    return pl.BlockSpec((2, H), lambda i: (0, 0))


def _row_spec():
    return pl.BlockSpec((BN, H), lambda i: (i, 0))


def _mlp_pair(x, u, v, wu, bu, wb, bb, wc, bc):
    n = x.shape[0]
    return pl.pallas_call(
        _mlp_pair_body,
        grid=(n // BN,),
        in_specs=[_row_spec(), _row_spec(), _row_spec(),
                  _w_spec(), _b_spec(), _w_spec(), _b_spec(),
                  _w_spec(), pl.BlockSpec((1, H), lambda i: (0, 0))],
        out_specs=_row_spec(),
        out_shape=jax.ShapeDtypeStruct((n, H), jnp.float32),
    )(x, u, v, wu, bu, wb, bb, wc, bc)


def _mlp_single(x, m, wm, bm, wx, bx, wc, bc, msg_first):
    n = x.shape[0]
    return pl.pallas_call(
        functools.partial(_mlp_single_body, msg_first),
        grid=(n // BN,),
        in_specs=[_row_spec(), _row_spec(),
                  _w_spec(), _b_spec(), _w_spec(), _b_spec(),
                  _w_spec(), pl.BlockSpec((1, H), lambda i: (0, 0))],
        out_specs=_row_spec(),
        out_shape=jax.ShapeDtypeStruct((n, H), jnp.float32),
    )(x, m, wm, bm, wx, bx, wc, bc)


def _readout_body(p0_ref, p1_ref, p2_ref, w1_ref, b1_ref, w2_ref, b2_ref, o_ref):
    t = jnp.maximum(jnp.dot(p0_ref[...], w1_ref[0], preferred_element_type=jnp.float32) + b1_ref[0], 0.0)
    t = t + jnp.maximum(jnp.dot(p1_ref[...], w1_ref[1], preferred_element_type=jnp.float32) + b1_ref[1], 0.0)
    t = t + jnp.maximum(jnp.dot(p2_ref[...], w1_ref[2], preferred_element_type=jnp.float32) + b1_ref[2], 0.0)
    o_ref[...] = jnp.dot(t, w2_ref[...], preferred_element_type=jnp.float32) + b2_ref[...]


def _readout(p0, p1, p2, w1, b1, w2, b2):
    return pl.pallas_call(
        _readout_body,
        grid=(NG // BN,),
        in_specs=[_row_spec(), _row_spec(), _row_spec(),
                  pl.BlockSpec((3, H, 2 * H), lambda i: (0, 0, 0)),
                  pl.BlockSpec((3, 2 * H), lambda i: (0, 0)),
                  pl.BlockSpec((2 * H, OUT), lambda i: (0, 0)),
                  pl.BlockSpec((1, OUT), lambda i: (0, 0))],
        out_specs=pl.BlockSpec((BN, OUT), lambda i: (i, 0)),
        out_shape=jax.ShapeDtypeStruct((NG, OUT), jnp.float32),
    )(p0, p1, p2, w1, b1, w2, b2)


def kernel(x0, up_index_0, up_index_1, boundary_index_1, boundary_index_2,
           batch0, batch1, batch2, atom_emb, W_up, b_up, g_up, bt_up,
           W_bd, b_bd, g_bd, bt_bd, W_cmb, b_cmb, g_cmb, bt_cmb,
           W1, b1, g1, bt1, W2, b2):
    f32 = jnp.float32
    Wup = W_up * g_up[:, :, :, None, :]
    bup = b_up * g_up + bt_up
    Wbd = W_bd * g_bd[:, :, :, None, :]
    bbd = b_bd * g_bd + bt_bd
    Wc = (W_cmb * g_cmb[:, :, None, :]).reshape(L, 3, 2, H, H)
    bc = (b_cmb * g_cmb + bt_cmb)[:, :, None, :]
    W1f = W1 * g1[:, None, :]
    b1f = (b1 * g1 + bt1)
    b1f2 = b1f
    W2f = W2
    b2f = b2[None, :]

    zeros = jnp.zeros((CMAX + 16, H), f32)

    tab_emb = atom_emb.reshape(9 * 100, H).astype(f32)
    codes = (x0.astype(jnp.int32)
             + jnp.arange(9, dtype=jnp.int32)[None, :] * 100).reshape(-1)
    emb_dst = jnp.repeat(jnp.arange(N0, dtype=jnp.int32), 9)
    emb_src, emb_dst = _pad_edges(codes, emb_dst, 900)
    nc0, C0 = CFG0
    est = [min(9 * C0 * k, 9 * N0) for k in range(nc0 + 1)]
    emb_starts = jnp.array(est + [9 * N0] * (16 - nc0 - 1), jnp.int32)
    x0f = _spmm(tab_emb, emb_src, emb_dst, emb_starts, zeros, CFG0)

    u0s, u0d, u0st = _group_edges(up_index_0[0], up_index_0[1], NP0, CFG0)
    u1s, u1d, u1st = _group_edges(up_index_1[0], up_index_1[1], NP1, CFG1)
    b1s, b1d, b1st = _group_edges(boundary_index_1[0], boundary_index_1[1], NP0, CFG1)
    b2s, b2d, b2st = _group_edges(boundary_index_2[0], boundary_index_2[1], NP1, CFG2)

    x1f = _spmm(x0f, b1s, b1d, b1st, zeros, CFG1)
    x2f = _spmm(x1f, b2s, b2d, b2st, zeros, CFG2)

    xs = [x0f, x1f, x2f]
    for l in range(L):
        up0 = _spmm(xs[0], u0s, u0d, u0st, zeros, CFG0)
        up1 = _spmm(xs[1], u1s, u1d, u1st, zeros, CFG1)
        bm1 = _spmm(xs[0], b1s, b1d, b1st, zeros, CFG1)
        bm2 = _spmm(xs[1], b2s, b2d, b2st, zeros, CFG2)
        x0n = _mlp_single(xs[0], up0, Wup[l, 0], bup[l, 0], Wbd[l, 0], bbd[l, 0],
                          Wc[l, 0], bc[l, 0], msg_first=True)
        x1n = _mlp_pair(xs[1], up1, bm1, Wup[l, 1], bup[l, 1], Wbd[l, 1], bbd[l, 1],
                        Wc[l, 1], bc[l, 1])
        x2n = _mlp_single(xs[2], bm2, Wbd[l, 2], bbd[l, 2], Wup[l, 2], bup[l, 2],
                          Wc[l, 2], bc[l, 2], msg_first=False)
        xs = [x0n, x1n, x2n]

    pools = []
    for d, (bat, n_d, npd) in enumerate(
            [(batch0, N0, NP0), (batch1, N1, NP1), (batch2, N2, NP2)]):
        psrc = jnp.arange(n_d, dtype=jnp.int32)
        pst = _sorted_starts(bat.astype(jnp.int32), CFGG, n_d)
        ps, pd_ = _pad_edges(psrc, bat.astype(jnp.int32), npd)
        pools.append(_spmm(xs[d], ps, pd_, pst, zeros, CFGG))

    return _readout(pools[0], pools[1], pools[2], W1f, b1f2, W2f, b2f)


# speedup vs baseline: 3.6835x; 1.0526x over previous
import functools

import jax
import jax.numpy as jnp
from jax import lax
from jax.experimental import pallas as pl
from jax.experimental.pallas import tpu as pltpu
from jax.experimental.pallas import tpu_sc as plsc

N0 = 100000
N1 = 160000
N2 = 20000
NG = 2048
L = 3
H = 64
OUT = 10

NB = 128
G = 4
RUN = G * NB
DMASK = (1 << 20) - 1
BIG = (1 << 29) | DMASK

CFG0 = (6, 16896)
CFG1 = (8, 20480)
CFG2 = (2, 10240)
CFGG = (2, 1024)
CMAX = 20480

NP0 = CFG0[0] * CFG0[1]
NP1 = CFG1[0] * CFG1[1]
NP2 = CFG2[0] * CFG2[1]


@functools.lru_cache(maxsize=None)
def _make_spmm(e_pad, nc, C, n_src):
    out_rows = nc * C
    mesh = plsc.VectorSubcoreMesh(core_axis_name="c", subcore_axis_name="s",
                                  num_cores=2, num_subcores=16)
    wr = C // 16

    def body(tab, srci, dsti, starts, zeros, out, sidx, didx, *rest):
        lidx = rest[:G]
        rows = rest[G:2 * G]
        starts_v = rest[2 * G]
        acc = rest[2 * G + 1]
        sem_si = rest[2 * G + 2]
        sem_di = rest[2 * G + 3]
        gsem = rest[2 * G + 4:3 * G + 4]
        sem_sc = rest[3 * G + 4]
        c = lax.axis_index("c")
        t = lax.axis_index("s")
        pltpu.sync_copy(starts, starts_v)
        sv = starts_v[...]
        for k in range(nc):
            @pl.when(c == (k % 2))
            def _chunk(k=k):
                base = k * C
                pltpu.sync_copy(zeros.at[pl.ds(t * wr, wr)],
                                acc.at[pl.ds(t * wr, wr)])
                plsc.subcore_barrier()
                s = sv[k]
                e = sv[k + 1]
                s8 = (s // 8) * 8

                @pl.loop((s8 + t * RUN).astype(jnp.int32), e, step=16 * RUN)
                def blk(off):
                    off = pl.multiple_of(off, 8)
                    cp1 = pltpu.async_copy(srci.at[pl.ds(off, RUN)], sidx, sem_si)
                    cp2 = pltpu.async_copy(dsti.at[pl.ds(off, RUN)], didx, sem_di)
                    cp1.wait()
                    cp2.wait()
                    for g in range(G):
                        for j in range(NB // 16):
                            dv = didx[pl.ds(g * NB + j * 16, 16)]
                            lv = (dv & DMASK) - base
                            ok = (lv >= 0) & (lv < C)
                            garb = C + lax.iota(jnp.int32, 16)
                            lidx[g][pl.ds(j * 16, 16)] = jnp.where(ok, lv, garb)
                    cps = [pltpu.async_copy(tab.at[sidx.at[pl.ds(g * NB, NB)]],
                                            rows[g], gsem[g]) for g in range(G)]
                    scs = []
                    for g in range(G):
                        cps[g].wait()
                        scs.append(pltpu.async_copy(rows[g], acc.at[lidx[g]],
                                                    sem_sc, add=True))
                    for sc in scs:
                        sc.wait()

                plsc.subcore_barrier()
                pltpu.sync_copy(acc.at[pl.ds(t * wr, wr)],
                                out.at[pl.ds(base + t * wr, wr)])

    return pl.kernel(
        body,
        out_type=jax.ShapeDtypeStruct((out_rows, H), jnp.float32),
        mesh=mesh,
        scratch_types=(
            [pltpu.VMEM((RUN,), jnp.int32)] * 2
            + [pltpu.VMEM((NB,), jnp.int32)] * G
            + [pltpu.VMEM((NB, H), jnp.float32)] * G
            + [pltpu.VMEM((16,), jnp.int32),
               pltpu.VMEM_SHARED((C + 16, H), jnp.float32)]
            + [pltpu.SemaphoreType.DMA] * (G + 3)
        ),
        compiler_params=pltpu.CompilerParams(use_tc_tiling_on_sc=False),
    )


def _spmm(tab, srci, dsti, starts16, zeros, cfg):
    nc, C = cfg
    k = _make_spmm(srci.shape[0], nc, C, tab.shape[0])
    return k(tab, srci, dsti, starts16, zeros)


def _pad_edges(src, dst, n_src):
    e = src.shape[0]
    e_pad = ((e + RUN - 1) // RUN + 1) * RUN
    npad = e_pad - e
    pad_src = (jnp.arange(npad, dtype=jnp.int32) * 67) % n_src
    src = jnp.concatenate([src.astype(jnp.int32), pad_src])
    dst = jnp.concatenate([dst.astype(jnp.int32),
                           jnp.full((npad,), BIG, jnp.int32)])
    return src, dst


def _group_edges(src, dst, n_src, cfg):
    nc, C = cfg
    e = src.shape[0]
    dst = dst.astype(jnp.int32)
    packed = ((dst // C) << 20) | dst
    sk, src_g = lax.sort((packed, src.astype(jnp.int32)), num_keys=1)
    bounds = jnp.arange(nc + 1, dtype=jnp.int32) << 20
    starts = jnp.searchsorted(sk, bounds).astype(jnp.int32)
    starts16 = jnp.concatenate([starts, jnp.full((16 - nc - 1,), e, jnp.int32)])
    src_g, dst_g = _pad_edges(src_g, sk, n_src)
    return src_g, dst_g, starts16


def _sorted_starts(dst_sorted, cfg, e):
    nc, C = cfg
    bounds = jnp.arange(nc + 1, dtype=jnp.int32) * C
    starts = jnp.searchsorted(dst_sorted, bounds).astype(jnp.int32)
    return jnp.concatenate([starts, jnp.full((16 - nc - 1,), e, jnp.int32)])


BN = 512


def _mlp_pair_body(x_ref, u_ref, v_ref, wu_ref, bu_ref, wb_ref, bb_ref,
                   wc_ref, bc_ref, o_ref):
    x = x_ref[...]
    a = x + u_ref[...]
    b = x + v_ref[...]
    a = jnp.maximum(jnp.dot(a, wu_ref[0], preferred_element_type=jnp.float32) + bu_ref[0], 0.0)
    a = jnp.maximum(jnp.dot(a, wu_ref[1], preferred_element_type=jnp.float32) + bu_ref[1], 0.0)
    b = jnp.maximum(jnp.dot(b, wb_ref[0], preferred_element_type=jnp.float32) + bb_ref[0], 0.0)
    b = jnp.maximum(jnp.dot(b, wb_ref[1], preferred_element_type=jnp.float32) + bb_ref[1], 0.0)
    h = (jnp.dot(a, wc_ref[0], preferred_element_type=jnp.float32)
         + jnp.dot(b, wc_ref[1], preferred_element_type=jnp.float32) + bc_ref[...])
    o_ref[...] = jnp.maximum(h, 0.0)


def _mlp_single_body(msg_first, x_ref, m_ref, wm_ref, bm_ref, wx_ref, bx_ref,
                     wc_ref, bc_ref, o_ref):
    x = x_ref[...]
    a = x + m_ref[...]
    a = jnp.maximum(jnp.dot(a, wm_ref[0], preferred_element_type=jnp.float32) + bm_ref[0], 0.0)
    a = jnp.maximum(jnp.dot(a, wm_ref[1], preferred_element_type=jnp.float32) + bm_ref[1], 0.0)
    b = jnp.maximum(jnp.dot(x, wx_ref[0], preferred_element_type=jnp.float32) + bx_ref[0], 0.0)
    b = jnp.maximum(jnp.dot(b, wx_ref[1], preferred_element_type=jnp.float32) + bx_ref[1], 0.0)
    if msg_first:
        h = (jnp.dot(a, wc_ref[0], preferred_element_type=jnp.float32)
             + jnp.dot(b, wc_ref[1], preferred_element_type=jnp.float32))
    else:
        h = (jnp.dot(b, wc_ref[0], preferred_element_type=jnp.float32)
             + jnp.dot(a, wc_ref[1], preferred_element_type=jnp.float32))
    o_ref[...] = jnp.maximum(h + bc_ref[...], 0.0)


def _w_spec():
    return pl.BlockSpec((2, H, H), lambda i: (0, 0, 0))


def _b_spec():
    return pl.BlockSpec((2, H), lambda i: (0, 0))


def _row_spec():
    return pl.BlockSpec((BN, H), lambda i: (i, 0))


def _mlp_pair(x, u, v, wu, bu, wb, bb, wc, bc):
    n = x.shape[0]
    return pl.pallas_call(
        _mlp_pair_body,
        grid=(n // BN,),
        in_specs=[_row_spec(), _row_spec(), _row_spec(),
                  _w_spec(), _b_spec(), _w_spec(), _b_spec(),
                  _w_spec(), pl.BlockSpec((1, H), lambda i: (0, 0))],
        out_specs=_row_spec(),
        out_shape=jax.ShapeDtypeStruct((n, H), jnp.float32),
    )(x, u, v, wu, bu, wb, bb, wc, bc)


def _mlp_single(x, m, wm, bm, wx, bx, wc, bc, msg_first):
    n = x.shape[0]
    return pl.pallas_call(
        functools.partial(_mlp_single_body, msg_first),
        grid=(n // BN,),
        in_specs=[_row_spec(), _row_spec(),
                  _w_spec(), _b_spec(), _w_spec(), _b_spec(),
                  _w_spec(), pl.BlockSpec((1, H), lambda i: (0, 0))],
        out_specs=_row_spec(),
        out_shape=jax.ShapeDtypeStruct((n, H), jnp.float32),
    )(x, m, wm, bm, wx, bx, wc, bc)


def _readout_body(p0_ref, p1_ref, p2_ref, w1_ref, b1_ref, w2_ref, b2_ref, o_ref):
    t = jnp.maximum(jnp.dot(p0_ref[...], w1_ref[0], preferred_element_type=jnp.float32) + b1_ref[0], 0.0)
    t = t + jnp.maximum(jnp.dot(p1_ref[...], w1_ref[1], preferred_element_type=jnp.float32) + b1_ref[1], 0.0)
    t = t + jnp.maximum(jnp.dot(p2_ref[...], w1_ref[2], preferred_element_type=jnp.float32) + b1_ref[2], 0.0)
    o_ref[...] = jnp.dot(t, w2_ref[...], preferred_element_type=jnp.float32) + b2_ref[...]


def _readout(p0, p1, p2, w1, b1, w2, b2):
    return pl.pallas_call(
        _readout_body,
        grid=(NG // BN,),
        in_specs=[_row_spec(), _row_spec(), _row_spec(),
                  pl.BlockSpec((3, H, 2 * H), lambda i: (0, 0, 0)),
                  pl.BlockSpec((3, 2 * H), lambda i: (0, 0)),
                  pl.BlockSpec((2 * H, OUT), lambda i: (0, 0)),
                  pl.BlockSpec((1, OUT), lambda i: (0, 0))],
        out_specs=pl.BlockSpec((BN, OUT), lambda i: (i, 0)),
        out_shape=jax.ShapeDtypeStruct((NG, OUT), jnp.float32),
    )(p0, p1, p2, w1, b1, w2, b2)


def kernel(x0, up_index_0, up_index_1, boundary_index_1, boundary_index_2,
           batch0, batch1, batch2, atom_emb, W_up, b_up, g_up, bt_up,
           W_bd, b_bd, g_bd, bt_bd, W_cmb, b_cmb, g_cmb, bt_cmb,
           W1, b1, g1, bt1, W2, b2):
    f32 = jnp.float32
    Wup = W_up * g_up[:, :, :, None, :]
    bup = b_up * g_up + bt_up
    Wbd = W_bd * g_bd[:, :, :, None, :]
    bbd = b_bd * g_bd + bt_bd
    Wc = (W_cmb * g_cmb[:, :, None, :]).reshape(L, 3, 2, H, H)
    bc = (b_cmb * g_cmb + bt_cmb)[:, :, None, :]
    W1f = W1 * g1[:, None, :]
    b1f = (b1 * g1 + bt1)
    b1f2 = b1f
    W2f = W2
    b2f = b2[None, :]

    zeros = jnp.zeros((CMAX + 16, H), f32)

    tab_emb = atom_emb.reshape(9 * 100, H).astype(f32)
    codes = (x0.astype(jnp.int32)
             + jnp.arange(9, dtype=jnp.int32)[None, :] * 100).reshape(-1)
    emb_dst = jnp.repeat(jnp.arange(N0, dtype=jnp.int32), 9)
    emb_src, emb_dst = _pad_edges(codes, emb_dst, 900)
    nc0, C0 = CFG0
    est = [min(9 * C0 * k, 9 * N0) for k in range(nc0 + 1)]
    emb_starts = jnp.array(est + [9 * N0] * (16 - nc0 - 1), jnp.int32)
    x0f = _spmm(tab_emb, emb_src, emb_dst, emb_starts, zeros, CFG0)

    u0s, u0d, u0st = _group_edges(up_index_0[0], up_index_0[1], NP0, CFG0)
    u1s, u1d, u1st = _group_edges(up_index_1[0], up_index_1[1], NP1, CFG1)
    b1s, b1d, b1st = _group_edges(boundary_index_1[0], boundary_index_1[1], NP0, CFG1)
    b2s, b2d, b2st = _group_edges(boundary_index_2[0], boundary_index_2[1], NP1, CFG2)

    x1f = _spmm(x0f, b1s, b1d, b1st, zeros, CFG1)
    x2f = _spmm(x1f, b2s, b2d, b2st, zeros, CFG2)

    xs = [x0f, x1f, x2f]
    for l in range(L):
        up0 = _spmm(xs[0], u0s, u0d, u0st, zeros, CFG0)
        up1 = _spmm(xs[1], u1s, u1d, u1st, zeros, CFG1)
        bm1 = _spmm(xs[0], b1s, b1d, b1st, zeros, CFG1)
        bm2 = _spmm(xs[1], b2s, b2d, b2st, zeros, CFG2)
        x0n = _mlp_single(xs[0], up0, Wup[l, 0], bup[l, 0], Wbd[l, 0], bbd[l, 0],
                          Wc[l, 0], bc[l, 0], msg_first=True)
        x1n = _mlp_pair(xs[1], up1, bm1, Wup[l, 1], bup[l, 1], Wbd[l, 1], bbd[l, 1],
                        Wc[l, 1], bc[l, 1])
        x2n = _mlp_single(xs[2], bm2, Wbd[l, 2], bbd[l, 2], Wup[l, 2], bup[l, 2],
                          Wc[l, 2], bc[l, 2], msg_first=False)
        xs = [x0n, x1n, x2n]

    pools = []
    for d, (bat, n_d, npd) in enumerate(
            [(batch0, N0, NP0), (batch1, N1, NP1), (batch2, N2, NP2)]):
        psrc = jnp.arange(n_d, dtype=jnp.int32)
        pst = _sorted_starts(bat.astype(jnp.int32), CFGG, n_d)
        ps, pd_ = _pad_edges(psrc, bat.astype(jnp.int32), npd)
        pools.append(_spmm(xs[d], ps, pd_, pst, zeros, CFGG))

    return _readout(pools[0], pools[1], pools[2], W1f, b1f2, W2f, b2f)


# speedup vs baseline: 3.6881x; 1.0012x over previous
import functools

import jax
import jax.numpy as jnp
from jax import lax
from jax.experimental import pallas as pl
from jax.experimental.pallas import tpu as pltpu
from jax.experimental.pallas import tpu_sc as plsc

N0 = 100000
N1 = 160000
N2 = 20000
NG = 2048
L = 3
H = 64
OUT = 10

NB = 128
G = 4
RUN = G * NB
DMASK = (1 << 20) - 1
BIG = (1 << 29) | DMASK

CFG0 = (6, 16896)
CFG1 = (8, 20480)
CFG2 = (2, 10240)
CFGG = (2, 1024)
CMAX = 20480

NP0 = CFG0[0] * CFG0[1]
NP1 = CFG1[0] * CFG1[1]
NP2 = CFG2[0] * CFG2[1]


@functools.lru_cache(maxsize=None)
def _make_spmm(e_pad, nc, C, n_src, stage_tab=False):
    out_rows = nc * C
    mesh = plsc.VectorSubcoreMesh(core_axis_name="c", subcore_axis_name="s",
                                  num_cores=2, num_subcores=16)
    wr = C // 16

    def body(tab, srci, dsti, starts, zeros, out, sidx, didx, *rest):
        lidx = rest[:G]
        rows = rest[G:2 * G]
        starts_v = rest[2 * G]
        acc = rest[2 * G + 1]
        sem_si = rest[2 * G + 2]
        sem_di = rest[2 * G + 3]
        gsem = rest[2 * G + 4:3 * G + 4]
        sem_sc = rest[3 * G + 4]
        c = lax.axis_index("c")
        t = lax.axis_index("s")
        pltpu.sync_copy(starts, starts_v)
        sv = starts_v[...]
        if stage_tab:
            tabs = rest[3 * G + 5]

            @pl.when(t == 0)
            def _stage():
                pltpu.sync_copy(tab, tabs)
            gtab = tabs
        else:
            gtab = tab
        for k in range(nc):
            @pl.when(c == (k % 2))
            def _chunk(k=k):
                base = k * C
                pltpu.sync_copy(zeros.at[pl.ds(t * wr, wr)],
                                acc.at[pl.ds(t * wr, wr)])
                plsc.subcore_barrier()
                s = sv[k]
                e = sv[k + 1]
                s8 = (s // 8) * 8

                @pl.loop((s8 + t * RUN).astype(jnp.int32), e, step=16 * RUN)
                def blk(off):
                    off = pl.multiple_of(off, 8)
                    cp1 = pltpu.async_copy(srci.at[pl.ds(off, RUN)], sidx, sem_si)
                    cp2 = pltpu.async_copy(dsti.at[pl.ds(off, RUN)], didx, sem_di)
                    cp1.wait()
                    cp2.wait()
                    for g in range(G):
                        for j in range(NB // 16):
                            dv = didx[pl.ds(g * NB + j * 16, 16)]
                            lv = (dv & DMASK) - base
                            ok = (lv >= 0) & (lv < C)
                            garb = C + lax.iota(jnp.int32, 16)
                            lidx[g][pl.ds(j * 16, 16)] = jnp.where(ok, lv, garb)
                    cps = [pltpu.async_copy(gtab.at[sidx.at[pl.ds(g * NB, NB)]],
                                            rows[g], gsem[g]) for g in range(G)]
                    scs = []
                    for g in range(G):
                        cps[g].wait()
                        scs.append(pltpu.async_copy(rows[g], acc.at[lidx[g]],
                                                    sem_sc, add=True))
                    for sc in scs:
                        sc.wait()

                plsc.subcore_barrier()
                pltpu.sync_copy(acc.at[pl.ds(t * wr, wr)],
                                out.at[pl.ds(base + t * wr, wr)])

    return pl.kernel(
        body,
        out_type=jax.ShapeDtypeStruct((out_rows, H), jnp.float32),
        mesh=mesh,
        scratch_types=(
            [pltpu.VMEM((RUN,), jnp.int32)] * 2
            + [pltpu.VMEM((NB,), jnp.int32)] * G
            + [pltpu.VMEM((NB, H), jnp.float32)] * G
            + [pltpu.VMEM((16,), jnp.int32),
               pltpu.VMEM_SHARED((C + 16, H), jnp.float32)]
            + [pltpu.SemaphoreType.DMA] * (G + 3)
            + ([pltpu.VMEM_SHARED((n_src, H), jnp.float32)] if stage_tab else [])
        ),
        compiler_params=pltpu.CompilerParams(use_tc_tiling_on_sc=False),
    )


def _spmm(tab, srci, dsti, starts16, zeros, cfg, stage_tab=False):
    nc, C = cfg
    k = _make_spmm(srci.shape[0], nc, C, tab.shape[0], stage_tab)
    return k(tab, srci, dsti, starts16, zeros)


def _pad_edges(src, dst, n_src):
    e = src.shape[0]
    e_pad = ((e + RUN - 1) // RUN + 1) * RUN
    npad = e_pad - e
    pad_src = (jnp.arange(npad, dtype=jnp.int32) * 67) % n_src
    src = jnp.concatenate([src.astype(jnp.int32), pad_src])
    dst = jnp.concatenate([dst.astype(jnp.int32),
                           jnp.full((npad,), BIG, jnp.int32)])
    return src, dst


def _group_edges(src, dst, n_src, cfg):
    nc, C = cfg
    e = src.shape[0]
    dst = dst.astype(jnp.int32)
    packed = ((dst // C) << 20) | dst
    sk, src_g = lax.sort((packed, src.astype(jnp.int32)), num_keys=1)
    bounds = jnp.arange(nc + 1, dtype=jnp.int32) << 20
    starts = jnp.searchsorted(sk, bounds).astype(jnp.int32)
    starts16 = jnp.concatenate([starts, jnp.full((16 - nc - 1,), e, jnp.int32)])
    src_g, dst_g = _pad_edges(src_g, sk, n_src)
    return src_g, dst_g, starts16


def _sorted_starts(dst_sorted, cfg, e):
    nc, C = cfg
    bounds = jnp.arange(nc + 1, dtype=jnp.int32) * C
    starts = jnp.searchsorted(dst_sorted, bounds).astype(jnp.int32)
    return jnp.concatenate([starts, jnp.full((16 - nc - 1,), e, jnp.int32)])


BN = 512


def _mlp_pair_body(x_ref, u_ref, v_ref, wu_ref, bu_ref, wb_ref, bb_ref,
                   wc_ref, bc_ref, o_ref):
    x = x_ref[...]
    a = x + u_ref[...]
    b = x + v_ref[...]
    a = jnp.maximum(jnp.dot(a, wu_ref[0], preferred_element_type=jnp.float32) + bu_ref[0], 0.0)
    a = jnp.maximum(jnp.dot(a, wu_ref[1], preferred_element_type=jnp.float32) + bu_ref[1], 0.0)
    b = jnp.maximum(jnp.dot(b, wb_ref[0], preferred_element_type=jnp.float32) + bb_ref[0], 0.0)
    b = jnp.maximum(jnp.dot(b, wb_ref[1], preferred_element_type=jnp.float32) + bb_ref[1], 0.0)
    h = (jnp.dot(a, wc_ref[0], preferred_element_type=jnp.float32)
         + jnp.dot(b, wc_ref[1], preferred_element_type=jnp.float32) + bc_ref[...])
    o_ref[...] = jnp.maximum(h, 0.0)


def _mlp_single_body(msg_first, x_ref, m_ref, wm_ref, bm_ref, wx_ref, bx_ref,
                     wc_ref, bc_ref, o_ref):
    x = x_ref[...]
    a = x + m_ref[...]
    a = jnp.maximum(jnp.dot(a, wm_ref[0], preferred_element_type=jnp.float32) + bm_ref[0], 0.0)
    a = jnp.maximum(jnp.dot(a, wm_ref[1], preferred_element_type=jnp.float32) + bm_ref[1], 0.0)
    b = jnp.maximum(jnp.dot(x, wx_ref[0], preferred_element_type=jnp.float32) + bx_ref[0], 0.0)
    b = jnp.maximum(jnp.dot(b, wx_ref[1], preferred_element_type=jnp.float32) + bx_ref[1], 0.0)
    if msg_first:
        h = (jnp.dot(a, wc_ref[0], preferred_element_type=jnp.float32)
             + jnp.dot(b, wc_ref[1], preferred_element_type=jnp.float32))
    else:
        h = (jnp.dot(b, wc_ref[0], preferred_element_type=jnp.float32)
             + jnp.dot(a, wc_ref[1], preferred_element_type=jnp.float32))
    o_ref[...] = jnp.maximum(h + bc_ref[...], 0.0)


def _w_spec():
    return pl.BlockSpec((2, H, H), lambda i: (0, 0, 0))


def _b_spec():
    return pl.BlockSpec((2, H), lambda i: (0, 0))


def _row_spec():
    return pl.BlockSpec((BN, H), lambda i: (i, 0))


def _mlp_pair(x, u, v, wu, bu, wb, bb, wc, bc):
    n = x.shape[0]
    return pl.pallas_call(
        _mlp_pair_body,
        grid=(n // BN,),
        in_specs=[_row_spec(), _row_spec(), _row_spec(),
                  _w_spec(), _b_spec(), _w_spec(), _b_spec(),
                  _w_spec(), pl.BlockSpec((1, H), lambda i: (0, 0))],
        out_specs=_row_spec(),
        out_shape=jax.ShapeDtypeStruct((n, H), jnp.float32),
    )(x, u, v, wu, bu, wb, bb, wc, bc)


def _mlp_single(x, m, wm, bm, wx, bx, wc, bc, msg_first):
    n = x.shape[0]
    return pl.pallas_call(
        functools.partial(_mlp_single_body, msg_first),
        grid=(n // BN,),
        in_specs=[_row_spec(), _row_spec(),
                  _w_spec(), _b_spec(), _w_spec(), _b_spec(),
                  _w_spec(), pl.BlockSpec((1, H), lambda i: (0, 0))],
        out_specs=_row_spec(),
        out_shape=jax.ShapeDtypeStruct((n, H), jnp.float32),
    )(x, m, wm, bm, wx, bx, wc, bc)


def _readout_body(p0_ref, p1_ref, p2_ref, w1_ref, b1_ref, w2_ref, b2_ref, o_ref):
    t = jnp.maximum(jnp.dot(p0_ref[...], w1_ref[0], preferred_element_type=jnp.float32) + b1_ref[0], 0.0)
    t = t + jnp.maximum(jnp.dot(p1_ref[...], w1_ref[1], preferred_element_type=jnp.float32) + b1_ref[1], 0.0)
    t = t + jnp.maximum(jnp.dot(p2_ref[...], w1_ref[2], preferred_element_type=jnp.float32) + b1_ref[2], 0.0)
    o_ref[...] = jnp.dot(t, w2_ref[...], preferred_element_type=jnp.float32) + b2_ref[...]


def _readout(p0, p1, p2, w1, b1, w2, b2):
    return pl.pallas_call(
        _readout_body,
        grid=(NG // BN,),
        in_specs=[_row_spec(), _row_spec(), _row_spec(),
                  pl.BlockSpec((3, H, 2 * H), lambda i: (0, 0, 0)),
                  pl.BlockSpec((3, 2 * H), lambda i: (0, 0)),
                  pl.BlockSpec((2 * H, OUT), lambda i: (0, 0)),
                  pl.BlockSpec((1, OUT), lambda i: (0, 0))],
        out_specs=pl.BlockSpec((BN, OUT), lambda i: (i, 0)),
        out_shape=jax.ShapeDtypeStruct((NG, OUT), jnp.float32),
    )(p0, p1, p2, w1, b1, w2, b2)


def kernel(x0, up_index_0, up_index_1, boundary_index_1, boundary_index_2,
           batch0, batch1, batch2, atom_emb, W_up, b_up, g_up, bt_up,
           W_bd, b_bd, g_bd, bt_bd, W_cmb, b_cmb, g_cmb, bt_cmb,
           W1, b1, g1, bt1, W2, b2):
    f32 = jnp.float32
    Wup = W_up * g_up[:, :, :, None, :]
    bup = b_up * g_up + bt_up
    Wbd = W_bd * g_bd[:, :, :, None, :]
    bbd = b_bd * g_bd + bt_bd
    Wc = (W_cmb * g_cmb[:, :, None, :]).reshape(L, 3, 2, H, H)
    bc = (b_cmb * g_cmb + bt_cmb)[:, :, None, :]
    W1f = W1 * g1[:, None, :]
    b1f = (b1 * g1 + bt1)
    b1f2 = b1f
    W2f = W2
    b2f = b2[None, :]

    zeros = jnp.zeros((CMAX + 16, H), f32)

    tab_emb = jnp.pad(atom_emb.reshape(9 * 100, H).astype(f32),
                      ((0, 12), (0, 0)))
    codes = (x0.astype(jnp.int32)
             + jnp.arange(9, dtype=jnp.int32)[None, :] * 100).reshape(-1)
    emb_dst = jnp.repeat(jnp.arange(N0, dtype=jnp.int32), 9)
    emb_src, emb_dst = _pad_edges(codes, emb_dst, 900)
    nc0, C0 = CFG0
    est = [min(9 * C0 * k, 9 * N0) for k in range(nc0 + 1)]
    emb_starts = jnp.array(est + [9 * N0] * (16 - nc0 - 1), jnp.int32)
    x0f = _spmm(tab_emb, emb_src, emb_dst, emb_starts, zeros, CFG0,
                stage_tab=True)

    u0s, u0d, u0st = _group_edges(up_index_0[0], up_index_0[1], NP0, CFG0)
    u1s, u1d, u1st = _group_edges(up_index_1[0], up_index_1[1], NP1, CFG1)
    b1s, b1d, b1st = _group_edges(boundary_index_1[0], boundary_index_1[1], NP0, CFG1)
    b2s, b2d, b2st = _group_edges(boundary_index_2[0], boundary_index_2[1], NP1, CFG2)

    x1f = _spmm(x0f, b1s, b1d, b1st, zeros, CFG1)
    x2f = _spmm(x1f, b2s, b2d, b2st, zeros, CFG2)

    xs = [x0f, x1f, x2f]
    for l in range(L):
        up0 = _spmm(xs[0], u0s, u0d, u0st, zeros, CFG0)
        up1 = _spmm(xs[1], u1s, u1d, u1st, zeros, CFG1)
        bm1 = _spmm(xs[0], b1s, b1d, b1st, zeros, CFG1)
        bm2 = _spmm(xs[1], b2s, b2d, b2st, zeros, CFG2)
        x0n = _mlp_single(xs[0], up0, Wup[l, 0], bup[l, 0], Wbd[l, 0], bbd[l, 0],
                          Wc[l, 0], bc[l, 0], msg_first=True)
        x1n = _mlp_pair(xs[1], up1, bm1, Wup[l, 1], bup[l, 1], Wbd[l, 1], bbd[l, 1],
                        Wc[l, 1], bc[l, 1])
        x2n = _mlp_single(xs[2], bm2, Wbd[l, 2], bbd[l, 2], Wup[l, 2], bup[l, 2],
                          Wc[l, 2], bc[l, 2], msg_first=False)
        xs = [x0n, x1n, x2n]

    pools = []
    for d, (bat, n_d, npd) in enumerate(
            [(batch0, N0, NP0), (batch1, N1, NP1), (batch2, N2, NP2)]):
        psrc = jnp.arange(n_d, dtype=jnp.int32)
        pst = _sorted_starts(bat.astype(jnp.int32), CFGG, n_d)
        ps, pd_ = _pad_edges(psrc, bat.astype(jnp.int32), npd)
        pools.append(_spmm(xs[d], ps, pd_, pst, zeros, CFGG))

    return _readout(pools[0], pools[1], pools[2], W1f, b1f2, W2f, b2f)
